# Initial kernel scaffold; baseline (speedup 1.0000x reference)
import jax
import jax.numpy as jnp
from jax.experimental import pallas as pl


def kernel(X, E_weight):
    raise NotImplementedError("write your pallas kernel here")


# speedup vs baseline: 4.6380x; 4.6380x over previous
import functools

import jax
import jax.numpy as jnp
from jax import lax
from jax.experimental import pallas as pl
from jax.experimental.pallas import tpu as pltpu
from jax.experimental.pallas import tpu_sc as plsc

_D = 64
_NC = 2
_NS = 16
_NW = _NC * _NS
_CHUNK = 128
_K = 4


@functools.cache
def _build(n_rows: int):
    b_per_w = n_rows // _NW
    nchunk = b_per_w // _CHUNK
    mesh = plsc.VectorSubcoreMesh(core_axis_name="c", subcore_axis_name="s")

    @functools.partial(
        pl.kernel,
        mesh=mesh,
        out_type=jax.ShapeDtypeStruct((n_rows, _D), jnp.float32),
        compiler_params=pltpu.CompilerParams(use_tc_tiling_on_sc=False),
        scratch_types=[
            pltpu.VMEM((nchunk, _CHUNK), jnp.int32),
            pltpu.VMEM((_K, _CHUNK, _D), jnp.float32),
            pltpu.SemaphoreType.DMA((_K,)),
            pltpu.SemaphoreType.DMA((_K,)),
        ],
    )
    def gather_kernel(idx_hbm, table_hbm, out_hbm, idx_v, rows_v, gsem, wsem):
        wid = lax.axis_index("s") * _NC + lax.axis_index("c")
        base = wid * b_per_w
        pltpu.sync_copy(idx_hbm.at[wid], idx_v)

        def start_gather(g):
            b = g % _K
            return pltpu.async_copy(
                table_hbm.at[idx_v.at[g]], rows_v.at[b], gsem.at[b]
            )

        def start_write(g):
            b = g % _K
            return pltpu.async_copy(
                rows_v.at[b],
                out_hbm.at[pl.ds(base + g * _CHUNK, _CHUNK)],
                wsem.at[b],
            )

        gathers = [None] * nchunk
        writes = [None] * nchunk
        waited = [False] * nchunk

        nprime = min(_K - 1, nchunk)
        for g in range(nprime):
            gathers[g] = start_gather(g)

        for g in range(nchunk):
            nxt = g + _K - 1
            if nprime <= nxt < nchunk:
                prev = nxt - _K
                if prev >= 0:
                    writes[prev].wait()
                    waited[prev] = True
                gathers[nxt] = start_gather(nxt)
            gathers[g].wait()
            writes[g] = start_write(g)

        for g in range(nchunk):
            if not waited[g]:
                writes[g].wait()

    return gather_kernel


def kernel(X, E_weight):
    n = X.size
    idx = X.reshape(_NW, n // (_NW * _CHUNK), _CHUNK).astype(jnp.int32)
    out = _build(n)(idx, E_weight)
    return out.reshape(X.shape + (E_weight.shape[1],))


# speedup vs baseline: 4.6555x; 1.0038x over previous
import functools

import jax
import jax.numpy as jnp
from jax import lax
from jax.experimental import pallas as pl
from jax.experimental.pallas import tpu as pltpu
from jax.experimental.pallas import tpu_sc as plsc

_D = 64
_NC = 2
_NS = 16
_NW = _NC * _NS
_CHUNK = 320
_K = 4


@functools.cache
def _build(n_rows: int):
    b_per_w = n_rows // _NW
    nchunk = b_per_w // _CHUNK
    mesh = plsc.VectorSubcoreMesh(core_axis_name="c", subcore_axis_name="s")

    @functools.partial(
        pl.kernel,
        mesh=mesh,
        out_type=jax.ShapeDtypeStruct((n_rows, _D), jnp.float32),
        compiler_params=pltpu.CompilerParams(use_tc_tiling_on_sc=False),
        scratch_types=[
            pltpu.VMEM((nchunk, _CHUNK), jnp.int32),
            pltpu.VMEM((_K, _CHUNK, _D), jnp.float32),
            pltpu.SemaphoreType.DMA((_K,)),
            pltpu.SemaphoreType.DMA((_K,)),
        ],
    )
    def gather_kernel(idx_hbm, table_hbm, out_hbm, idx_v, rows_v, gsem, wsem):
        wid = lax.axis_index("s") * _NC + lax.axis_index("c")
        base = wid * b_per_w
        pltpu.sync_copy(idx_hbm.at[wid], idx_v)

        def start_gather(g):
            b = g % _K
            return pltpu.async_copy(
                table_hbm.at[idx_v.at[g]], rows_v.at[b], gsem.at[b]
            )

        def start_write(g):
            b = g % _K
            return pltpu.async_copy(
                rows_v.at[b],
                out_hbm.at[pl.ds(base + g * _CHUNK, _CHUNK)],
                wsem.at[b],
            )

        gathers = [None] * nchunk
        writes = [None] * nchunk
        waited = [False] * nchunk

        nprime = min(_K - 1, nchunk)
        for g in range(nprime):
            gathers[g] = start_gather(g)

        for g in range(nchunk):
            nxt = g + _K - 1
            if nprime <= nxt < nchunk:
                prev = nxt - _K
                if prev >= 0:
                    writes[prev].wait()
                    waited[prev] = True
                gathers[nxt] = start_gather(nxt)
            gathers[g].wait()
            writes[g] = start_write(g)

        for g in range(nchunk):
            if not waited[g]:
                writes[g].wait()

    return gather_kernel


def kernel(X, E_weight):
    n = X.size
    idx = X.reshape(_NW, n // (_NW * _CHUNK), _CHUNK).astype(jnp.int32)
    out = _build(n)(idx, E_weight)
    return out.reshape(X.shape + (E_weight.shape[1],))
